# Initial kernel scaffold; baseline (speedup 1.0000x reference)
import jax
import jax.numpy as jnp
from jax.experimental import pallas as pl


def kernel(mi_sim, me_sim, edge_index, W_mi, W_me):
    raise NotImplementedError("write your pallas kernel here")


# speedup vs baseline: 4.5004x; 4.5004x over previous
import functools

import jax
import jax.numpy as jnp
from jax import lax
from jax.experimental import pallas as pl
from jax.experimental.pallas import tpu as pltpu
from jax.experimental.pallas import tpu_sc as plsc

N_MI = 5000
N = 10000
D = 512
F = 128
E = 320000
SLOPE = 0.2

L = 16
NC = 2
NS = 16
NW = NC * NS
NPAD = 10240
EPT = E // NW
CH = 80
NCHUNK = EPT // CH
BPC = CH // L
CPT = NPAD // NS
NEG = -1e30


def _rot_perms():
    iota = lax.iota(jnp.int32, L)
    return [(iota + s) & (L - 1) for s in range(1, L)]


def _take(x, idx):
    return jnp.take_along_axis(x, idx, axis=0, mode="promise_in_bounds")


def _matmul_body(x_ref, w_ref, o_ref):
    o_ref[...] = jnp.dot(
        x_ref[...], w_ref[0],
        preferred_element_type=jnp.float32,
    )


def _compute_z(xall, wstack):
    BR = 1000
    return pl.pallas_call(
        _matmul_body,
        grid=(N // BR,),
        in_specs=[
            pl.BlockSpec((BR, D), lambda i: (i, 0)),
            pl.BlockSpec((1, D, F), lambda i: (i * BR // N_MI, 0, 0)),
        ],
        out_specs=pl.BlockSpec((BR, F), lambda i: (i, 0)),
        out_shape=jax.ShapeDtypeStruct((N, F), jnp.float32),
    )(xall, wstack)


def _edge_logits(z, srcv, dstv):
    mesh = plsc.VectorSubcoreMesh(core_axis_name="c", subcore_axis_name="s")

    @functools.partial(
        pl.kernel,
        out_type=(
            jax.ShapeDtypeStruct((E,), jnp.float32),
            jax.ShapeDtypeStruct((NW, NPAD), jnp.float32),
        ),
        mesh=mesh,
        scratch_types=[
            pltpu.VMEM((CH,), jnp.int32),
            pltpu.VMEM((CH,), jnp.int32),
            pltpu.VMEM((CH, F), jnp.float32),
            pltpu.VMEM((CH, F), jnp.float32),
            pltpu.VMEM((CH,), jnp.float32),
            pltpu.VMEM((L, L), jnp.float32),
            pltpu.VMEM((NPAD,), jnp.float32),
            pltpu.SemaphoreType.DMA,
            pltpu.SemaphoreType.DMA,
        ],
        compiler_params=pltpu.CompilerParams(needs_layout_passes=False),
    )
    def k(z_hbm, src_hbm, dst_hbm, e_hbm, pmax_hbm,
          sidx, didx, zs, zd, ebuf, scr, pmax_v, sem1, sem2):
        cid = lax.axis_index("c")
        sid = lax.axis_index("s")
        wid = sid * NC + cid
        base = wid * EPT
        perms = _rot_perms()
        iota = lax.iota(jnp.int32, L)

        def init_body(i, carry):
            pmax_v[pl.ds(i * L, L)] = jnp.full((L,), NEG, jnp.float32)
            return carry
        lax.fori_loop(0, NPAD // L, init_body, 0)

        def chunk_body(k0, carry):
            off = base + k0 * CH
            pltpu.sync_copy(src_hbm.at[pl.ds(off, CH)], sidx)
            pltpu.sync_copy(dst_hbm.at[pl.ds(off, CH)], didx)
            pltpu.async_copy(z_hbm.at[sidx], zs, sem1).wait()
            pltpu.async_copy(z_hbm.at[didx], zd, sem2).wait()

            def block_body(j, bc):
                eb = j * L
                for t in range(L):
                    r = eb + t
                    acc = zs[r, pl.ds(0, L)] * zd[r, pl.ds(0, L)]
                    for c in range(1, F // L):
                        acc = acc + zs[r, pl.ds(c * L, L)] * zd[r, pl.ds(c * L, L)]
                    scr[t] = acc
                dots = plsc.load_gather(scr, [iota, jnp.full((L,), 0, jnp.int32)])
                for l in range(1, L):
                    dots = dots + plsc.load_gather(
                        scr, [iota, jnp.full((L,), l, jnp.int32)])
                ev = jnp.where(dots > 0, dots, SLOPE * dots)
                ebuf[pl.ds(eb, L)] = ev
                key = didx[pl.ds(eb, L)]
                gmax = ev
                for perm in perms:
                    kr = _take(key, perm)
                    vr = _take(ev, perm)
                    gmax = jnp.maximum(gmax, jnp.where(kr == key, vr, NEG))
                old = plsc.load_gather(pmax_v, [key])
                plsc.store_scatter(pmax_v, [key], jnp.maximum(old, gmax))
                return bc
            lax.fori_loop(0, BPC, block_body, 0)
            pltpu.sync_copy(ebuf, e_hbm.at[pl.ds(off, CH)])
            return carry
        lax.fori_loop(0, NCHUNK, chunk_body, 0)
        pltpu.sync_copy(pmax_v, pmax_hbm.at[wid])

    return k(z, srcv, dstv)


FH = F // 2


def _aggregate(z0, z1, srcv, dstv, e_arr, pmax):
    mesh = plsc.VectorSubcoreMesh(core_axis_name="c", subcore_axis_name="s")

    @functools.partial(
        pl.kernel,
        out_type=(
            jax.ShapeDtypeStruct((NC, 2, NPAD, FH), jnp.float32),
            jax.ShapeDtypeStruct((NW, NPAD), jnp.float32),
        ),
        mesh=mesh,
        scratch_types=[
            pltpu.VMEM((NW, CPT), jnp.float32),
            pltpu.VMEM((CPT,), jnp.float32),
            pltpu.VMEM((NPAD,), jnp.float32),
            pltpu.VMEM((NPAD,), jnp.float32),
            pltpu.VMEM((EPT,), jnp.float32),
            pltpu.VMEM((CH,), jnp.int32),
            pltpu.VMEM((CH,), jnp.int32),
            pltpu.VMEM((CH,), jnp.float32),
            pltpu.VMEM((CH, FH), jnp.float32),
            pltpu.VMEM((CH, FH), jnp.float32),
            pltpu.VMEM_SHARED((NPAD,), jnp.float32),
            pltpu.VMEM_SHARED((NPAD, FH), jnp.float32),
            pltpu.SemaphoreType.DMA,
        ],
        compiler_params=pltpu.CompilerParams(
            needs_layout_passes=False, use_tc_tiling_on_sc=False),
    )
    def k(z0_hbm, z1_hbm, src_hbm, dst_hbm, e_hbm, pmax_hbm, hp_hbm, pden_hbm,
          mtmp, msl, mx_full, pden_v, exal, sidx, didx, ebuf, zs, zbuf,
          mx_sh, h_sh, sem1):
        cid = lax.axis_index("c")
        sid = lax.axis_index("s")
        wid = sid * NC + cid
        base = wid * EPT
        col0 = sid * CPT
        perms = _rot_perms()

        for r in range(NW):
            pltpu.sync_copy(pmax_hbm.at[r, pl.ds(col0, CPT)], mtmp.at[r])

        def mred(v, carry):
            m = mtmp[0, pl.ds(v * L, L)]
            for r in range(1, NW):
                m = jnp.maximum(m, mtmp[r, pl.ds(v * L, L)])
            msl[pl.ds(v * L, L)] = m
            return carry
        lax.fori_loop(0, CPT // L, mred, 0)
        pltpu.sync_copy(msl, mx_sh.at[pl.ds(col0, CPT)])

        def zrow(i, carry):
            for c in range(FH // L):
                zbuf[i, pl.ds(c * L, L)] = jnp.zeros((L,), jnp.float32)
            return carry
        lax.fori_loop(0, CH, zrow, 0)

        def zden(i, carry):
            pden_v[pl.ds(i * L, L)] = jnp.zeros((L,), jnp.float32)
            return carry
        lax.fori_loop(0, NPAD // L, zden, 0)

        plsc.subcore_barrier()
        pltpu.sync_copy(mx_sh, mx_full)

        for half in range(2):
            zh_hbm = z0_hbm if half == 0 else z1_hbm
            for b in range(CPT // CH):
                pltpu.sync_copy(zbuf, h_sh.at[pl.ds(col0 + b * CH, CH)])
            plsc.subcore_barrier()

            def chunk_body(k0, carry):
                off = base + k0 * CH
                loc = k0 * CH
                pltpu.sync_copy(src_hbm.at[pl.ds(off, CH)], sidx)
                pltpu.sync_copy(dst_hbm.at[pl.ds(off, CH)], didx)
                if half == 0:
                    pltpu.sync_copy(e_hbm.at[pl.ds(off, CH)], ebuf)
                pltpu.async_copy(zh_hbm.at[sidx], zs, sem1).wait()

                def block_body(j, bc):
                    eb = j * L
                    key = didx[pl.ds(eb, L)]
                    if half == 0:
                        ev = ebuf[pl.ds(eb, L)]
                        mx = plsc.load_gather(mx_full, [key])
                        ex = jnp.exp(ev - mx)
                        exal[pl.ds(loc + eb, L)] = ex
                        tot = ex
                        for perm in perms:
                            kr = _take(key, perm)
                            vr = _take(ex, perm)
                            tot = tot + jnp.where(kr == key, vr, 0.0)
                        old = plsc.load_gather(pden_v, [key])
                        plsc.store_scatter(pden_v, [key], old + tot)
                    else:
                        ex = exal[pl.ds(loc + eb, L)]
                    for t in range(L):
                        r = eb + t
                        al = _take(ex, jnp.full((L,), t, jnp.int32))
                        for c in range(FH // L):
                            zs[r, pl.ds(c * L, L)] = zs[r, pl.ds(c * L, L)] * al
                    return bc
                lax.fori_loop(0, BPC, block_body, 0)
                pltpu.sync_copy(zs, h_sh.at[didx], add=True)
                return carry
            lax.fori_loop(0, NCHUNK, chunk_body, 0)

            plsc.subcore_barrier()
            pltpu.sync_copy(h_sh.at[pl.ds(col0, CPT)],
                            hp_hbm.at[cid, half, pl.ds(col0, CPT)])

        pltpu.sync_copy(pden_v, pden_hbm.at[wid])

    return k(z0, z1, srcv, dstv, e_arr, pmax)


def _finalize(hp, pden):
    BR = 2048

    def body(h_ref, d_ref, o_ref):
        den = jnp.sum(d_ref[...], axis=0)[:, None] + 1e-9
        for half in range(2):
            val = (h_ref[0, half] + h_ref[1, half]) / den
            o_ref[:, half * FH:(half + 1) * FH] = jnp.where(
                val > 0, val, jnp.exp(val) - 1.0)

    return pl.pallas_call(
        body,
        grid=(NPAD // BR,),
        in_specs=[
            pl.BlockSpec((NC, 2, BR, FH), lambda i: (0, 0, i, 0)),
            pl.BlockSpec((NW, BR), lambda i: (0, i)),
        ],
        out_specs=pl.BlockSpec((BR, F), lambda i: (i, 0)),
        out_shape=jax.ShapeDtypeStruct((NPAD, F), jnp.float32),
    )(hp, pden)


def kernel(mi_sim, me_sim, edge_index, W_mi, W_me):
    xall = jnp.concatenate([mi_sim, me_sim], axis=0)
    wstack = jnp.stack([W_mi, W_me])
    z = _compute_z(xall, wstack)
    src = edge_index[0].astype(jnp.int32)
    dst = edge_index[1].astype(jnp.int32)
    e_arr, pmax = _edge_logits(z, src, dst)
    hp, pden = _aggregate(z[:, :FH], z[:, FH:], src, dst, e_arr, pmax)
    return _finalize(hp, pden)[:N]


# speedup vs baseline: 9.2783x; 2.0617x over previous
import functools

import jax
import jax.numpy as jnp
from jax import lax
from jax.experimental import pallas as pl
from jax.experimental.pallas import tpu as pltpu
from jax.experimental.pallas import tpu_sc as plsc

N_MI = 5000
N = 10000
D = 512
F = 128
E = 320000
SLOPE = 0.2

L = 16
NC = 2
NS = 16
NW = NC * NS
NPAD = 10240
EPT = E // NW
CH = 80
NCHUNK = EPT // CH
BPC = CH // L
CPT = NPAD // NS
NEG = -1e30


def _rot_perms():
    iota = lax.iota(jnp.int32, L)
    return [(iota + s) & (L - 1) for s in range(1, L)]


def _take(x, idx):
    return jnp.take_along_axis(x, idx, axis=0, mode="promise_in_bounds")


def _matmul_body(x_ref, w_ref, o_ref):
    o_ref[...] = jnp.dot(
        x_ref[...], w_ref[0],
        preferred_element_type=jnp.float32,
    )


def _compute_z(xall, wstack):
    BR = 1000
    return pl.pallas_call(
        _matmul_body,
        grid=(N // BR,),
        in_specs=[
            pl.BlockSpec((BR, D), lambda i: (i, 0)),
            pl.BlockSpec((1, D, F), lambda i: (i * BR // N_MI, 0, 0)),
        ],
        out_specs=pl.BlockSpec((BR, F), lambda i: (i, 0)),
        out_shape=jax.ShapeDtypeStruct((N, F), jnp.float32),
    )(xall, wstack)


def _edge_logits(z, srcv, dstv):
    mesh = plsc.VectorSubcoreMesh(core_axis_name="c", subcore_axis_name="s")

    @functools.partial(
        pl.kernel,
        out_type=(
            jax.ShapeDtypeStruct((E,), jnp.float32),
            jax.ShapeDtypeStruct((NW, NPAD), jnp.float32),
        ),
        mesh=mesh,
        scratch_types=[
            [pltpu.VMEM((CH,), jnp.int32)] * 2,
            [pltpu.VMEM((CH,), jnp.int32)] * 2,
            [pltpu.VMEM((CH, F), jnp.float32)] * 2,
            [pltpu.VMEM((CH, F), jnp.float32)] * 2,
            [pltpu.VMEM((CH,), jnp.float32)] * 2,
            pltpu.VMEM((L, L), jnp.float32),
            pltpu.VMEM((NPAD,), jnp.float32),
            [pltpu.SemaphoreType.DMA] * 2,
            [pltpu.SemaphoreType.DMA] * 2,
            [pltpu.SemaphoreType.DMA] * 2,
        ],
        compiler_params=pltpu.CompilerParams(needs_layout_passes=False),
    )
    def k(z_hbm, src_hbm, dst_hbm, e_hbm, pmax_hbm,
          sidx, didx, zs, zd, ebuf, scr, pmax_v, semI, semZ, semE):
        cid = lax.axis_index("c")
        sid = lax.axis_index("s")
        wid = sid * NC + cid
        base = wid * EPT
        perms = _rot_perms()
        iota = lax.iota(jnp.int32, L)

        def init_body(i, carry):
            pmax_v[pl.ds(i * L, L)] = jnp.full((L,), NEG, jnp.float32)
            return carry
        lax.fori_loop(0, NPAD // L, init_body, 0)

        def issue_idx(kc, b):
            off = base + kc * CH
            pltpu.async_copy(src_hbm.at[pl.ds(off, CH)], sidx[b], semI[b])
            pltpu.async_copy(dst_hbm.at[pl.ds(off, CH)], didx[b], semI[b])

        def wait_idx(b):
            pltpu.make_async_copy(src_hbm.at[pl.ds(0, CH)], sidx[b], semI[b]).wait()
            pltpu.make_async_copy(dst_hbm.at[pl.ds(0, CH)], didx[b], semI[b]).wait()

        def issue_gather(b):
            pltpu.async_copy(z_hbm.at[sidx[b]], zs[b], semZ[b])
            pltpu.async_copy(z_hbm.at[didx[b]], zd[b], semZ[b])

        def wait_gather(b):
            pltpu.make_async_copy(z_hbm.at[sidx[b]], zs[b], semZ[b]).wait()
            pltpu.make_async_copy(z_hbm.at[didx[b]], zd[b], semZ[b]).wait()

        def compute_chunk(kc, b):
            def block_body(j, bc):
                eb = j * L
                for t in range(L):
                    r = eb + t
                    acc = zs[b][r, pl.ds(0, L)] * zd[b][r, pl.ds(0, L)]
                    for c in range(1, F // L):
                        acc = acc + (zs[b][r, pl.ds(c * L, L)]
                                     * zd[b][r, pl.ds(c * L, L)])
                    scr[t] = acc
                dots = plsc.load_gather(scr, [iota, jnp.full((L,), 0, jnp.int32)])
                for l in range(1, L):
                    dots = dots + plsc.load_gather(
                        scr, [iota, jnp.full((L,), l, jnp.int32)])
                ev = jnp.where(dots > 0, dots, SLOPE * dots)
                ebuf[b][pl.ds(eb, L)] = ev
                key = didx[b][pl.ds(eb, L)]
                gmax = ev
                for perm in perms:
                    kr = _take(key, perm)
                    vr = _take(ev, perm)
                    gmax = jnp.maximum(gmax, jnp.where(kr == key, vr, NEG))
                old = plsc.load_gather(pmax_v, [key])
                plsc.store_scatter(pmax_v, [key], jnp.maximum(old, gmax))
                return bc
            lax.fori_loop(0, BPC, block_body, 0)
            pltpu.async_copy(ebuf[b], e_hbm.at[pl.ds(base + kc * CH, CH)], semE[b])

        def process(kc, b, wait_e, next_gather, idx2):
            wait_gather(b)
            if next_gather:
                wait_idx(1 - b)
                issue_gather(1 - b)
            if wait_e:
                pltpu.make_async_copy(
                    ebuf[b], e_hbm.at[pl.ds(0, CH)], semE[b]).wait()
            compute_chunk(kc, b)
            if idx2 == "always":
                issue_idx(kc + 2, b)
            elif idx2 == "guard":
                @pl.when(kc + 2 < NCHUNK)
                def _():
                    issue_idx(kc + 2, b)

        pltpu.sync_copy(src_hbm.at[pl.ds(base, CH)], sidx[0])
        pltpu.sync_copy(dst_hbm.at[pl.ds(base, CH)], didx[0])
        issue_gather(0)
        issue_idx(1, 1)
        process(0, 0, False, True, "always")
        process(1, 1, False, True, "always")

        @pl.loop(2, NCHUNK - 1, step=2)
        def _(kc):
            process(kc, 0, True, True, "always")
            process(kc + 1, 1, True, True, "guard")

        process(NCHUNK - 1, 0, True, False, "skip")
        pltpu.make_async_copy(ebuf[0], e_hbm.at[pl.ds(0, CH)], semE[0]).wait()
        pltpu.make_async_copy(ebuf[1], e_hbm.at[pl.ds(0, CH)], semE[1]).wait()
        pltpu.sync_copy(pmax_v, pmax_hbm.at[wid])

    return k(z, srcv, dstv)


FH = F // 2


def _aggregate(z0, z1, srcv, dstv, e_arr, pmax):
    mesh = plsc.VectorSubcoreMesh(core_axis_name="c", subcore_axis_name="s")

    @functools.partial(
        pl.kernel,
        out_type=(
            jax.ShapeDtypeStruct((NC, 2, NPAD, FH), jnp.float32),
            jax.ShapeDtypeStruct((NW, NPAD), jnp.float32),
        ),
        mesh=mesh,
        scratch_types=[
            pltpu.VMEM((NW, CPT), jnp.float32),
            pltpu.VMEM((CPT,), jnp.float32),
            pltpu.VMEM((NPAD,), jnp.float32),
            pltpu.VMEM((NPAD,), jnp.float32),
            pltpu.VMEM((EPT,), jnp.float32),
            [pltpu.VMEM((CH,), jnp.int32)] * 2,
            [pltpu.VMEM((CH,), jnp.int32)] * 2,
            [pltpu.VMEM((CH,), jnp.int32)] * 2,
            [pltpu.VMEM((CH,), jnp.float32)] * 2,
            [pltpu.VMEM((CH, FH), jnp.float32)] * 2,
            pltpu.VMEM((CH, FH), jnp.float32),
            pltpu.VMEM_SHARED((NPAD,), jnp.float32),
            pltpu.VMEM_SHARED((NPAD, FH), jnp.float32),
            [pltpu.SemaphoreType.DMA] * 2,
            [pltpu.SemaphoreType.DMA] * 2,
            [pltpu.SemaphoreType.DMA] * 2,
        ],
        compiler_params=pltpu.CompilerParams(
            needs_layout_passes=False, use_tc_tiling_on_sc=False),
    )
    def k(z0_hbm, z1_hbm, src_hbm, dst_hbm, e_hbm, pmax_hbm, hp_hbm, pden_hbm,
          mtmp, msl, mx_full, pden_v, exal, sidx, didx, sdidx, ebuf, zs, zbuf,
          mx_sh, h_sh, semI, semZ, semA):
        cid = lax.axis_index("c")
        sid = lax.axis_index("s")
        wid = sid * NC + cid
        base = wid * EPT
        col0 = sid * CPT
        perms = _rot_perms()

        for r in range(NW):
            pltpu.sync_copy(pmax_hbm.at[r, pl.ds(col0, CPT)], mtmp.at[r])

        def mred(v, carry):
            m = mtmp[0, pl.ds(v * L, L)]
            for r in range(1, NW):
                m = jnp.maximum(m, mtmp[r, pl.ds(v * L, L)])
            msl[pl.ds(v * L, L)] = m
            return carry
        lax.fori_loop(0, CPT // L, mred, 0)
        pltpu.sync_copy(msl, mx_sh.at[pl.ds(col0, CPT)])

        def zrow(i, carry):
            for c in range(FH // L):
                zbuf[i, pl.ds(c * L, L)] = jnp.zeros((L,), jnp.float32)
            return carry
        lax.fori_loop(0, CH, zrow, 0)

        def zden(i, carry):
            pden_v[pl.ds(i * L, L)] = jnp.zeros((L,), jnp.float32)
            return carry
        lax.fori_loop(0, NPAD // L, zden, 0)

        plsc.subcore_barrier()
        pltpu.sync_copy(mx_sh, mx_full)

        for half in range(2):
            zh_hbm = z0_hbm if half == 0 else z1_hbm
            for b in range(CPT // CH):
                pltpu.sync_copy(zbuf, h_sh.at[pl.ds(col0 + b * CH, CH)])
            plsc.subcore_barrier()

            def issue_idx(kc, b):
                off = base + kc * CH
                pltpu.async_copy(src_hbm.at[pl.ds(off, CH)], sidx[b], semI[b])
                pltpu.async_copy(dst_hbm.at[pl.ds(off, CH)], didx[b], semI[b])
                if half == 0:
                    pltpu.async_copy(e_hbm.at[pl.ds(off, CH)], ebuf[b], semI[b])

            def wait_idx(b):
                pltpu.make_async_copy(
                    src_hbm.at[pl.ds(0, CH)], sidx[b], semI[b]).wait()
                pltpu.make_async_copy(
                    dst_hbm.at[pl.ds(0, CH)], didx[b], semI[b]).wait()
                if half == 0:
                    pltpu.make_async_copy(
                        e_hbm.at[pl.ds(0, CH)], ebuf[b], semI[b]).wait()

            def issue_gather(b):
                pltpu.async_copy(zh_hbm.at[sidx[b]], zs[b], semZ[b])

            def wait_gather(b):
                pltpu.make_async_copy(zh_hbm.at[sidx[b]], zs[b], semZ[b]).wait()

            def drain_scatter(b):
                pltpu.make_async_copy(
                    zs[b], h_sh.at[sdidx[b]], semA[b]).wait()

            def compute_chunk(kc, b):
                loc = kc * CH
                for v in range(CH // L):
                    sdidx[b][pl.ds(v * L, L)] = didx[b][pl.ds(v * L, L)]

                def block_body(j, bc):
                    eb = j * L
                    key = didx[b][pl.ds(eb, L)]
                    if half == 0:
                        ev = ebuf[b][pl.ds(eb, L)]
                        mx = plsc.load_gather(mx_full, [key])
                        ex = jnp.exp(ev - mx)
                        exal[pl.ds(loc + eb, L)] = ex
                        tot = ex
                        for perm in perms:
                            kr = _take(key, perm)
                            vr = _take(ex, perm)
                            tot = tot + jnp.where(kr == key, vr, 0.0)
                        old = plsc.load_gather(pden_v, [key])
                        plsc.store_scatter(pden_v, [key], old + tot)
                    else:
                        ex = exal[pl.ds(loc + eb, L)]
                    for t in range(L):
                        r = eb + t
                        al = _take(ex, jnp.full((L,), t, jnp.int32))
                        for c in range(FH // L):
                            zs[b][r, pl.ds(c * L, L)] = (
                                zs[b][r, pl.ds(c * L, L)] * al)
                    return bc
                lax.fori_loop(0, BPC, block_body, 0)
                pltpu.async_copy(zs[b], h_sh.at[sdidx[b]], semA[b], add=True)

            def process(kc, b, next_gather, drain_a, idx2):
                wait_gather(b)
                if next_gather:
                    if drain_a:
                        drain_scatter(1 - b)
                    wait_idx(1 - b)
                    issue_gather(1 - b)
                compute_chunk(kc, b)
                if idx2 == "always":
                    issue_idx(kc + 2, b)
                elif idx2 == "guard":
                    @pl.when(kc + 2 < NCHUNK)
                    def _():
                        issue_idx(kc + 2, b)

            pltpu.sync_copy(src_hbm.at[pl.ds(base, CH)], sidx[0])
            pltpu.sync_copy(dst_hbm.at[pl.ds(base, CH)], didx[0])
            if half == 0:
                pltpu.sync_copy(e_hbm.at[pl.ds(base, CH)], ebuf[0])
            issue_gather(0)
            issue_idx(1, 1)
            process(0, 0, True, False, "always")
            process(1, 1, True, True, "always")

            @pl.loop(2, NCHUNK - 1, step=2)
            def _(kc):
                process(kc, 0, True, True, "always")
                process(kc + 1, 1, True, True, "guard")

            process(NCHUNK - 1, 0, False, False, "skip")
            drain_scatter(0)
            drain_scatter(1)

            plsc.subcore_barrier()
            pltpu.sync_copy(h_sh.at[pl.ds(col0, CPT)],
                            hp_hbm.at[cid, half, pl.ds(col0, CPT)])
            plsc.subcore_barrier()

        pltpu.sync_copy(pden_v, pden_hbm.at[wid])

    return k(z0, z1, srcv, dstv, e_arr, pmax)


def _finalize(hp, pden):
    BR = 2048

    def body(h_ref, d_ref, o_ref):
        den = jnp.sum(d_ref[...], axis=0)[:, None] + 1e-9
        for half in range(2):
            val = (h_ref[0, half] + h_ref[1, half]) / den
            o_ref[:, half * FH:(half + 1) * FH] = jnp.where(
                val > 0, val, jnp.exp(val) - 1.0)

    return pl.pallas_call(
        body,
        grid=(NPAD // BR,),
        in_specs=[
            pl.BlockSpec((NC, 2, BR, FH), lambda i: (0, 0, i, 0)),
            pl.BlockSpec((NW, BR), lambda i: (0, i)),
        ],
        out_specs=pl.BlockSpec((BR, F), lambda i: (i, 0)),
        out_shape=jax.ShapeDtypeStruct((NPAD, F), jnp.float32),
    )(hp, pden)


def kernel(mi_sim, me_sim, edge_index, W_mi, W_me):
    xall = jnp.concatenate([mi_sim, me_sim], axis=0)
    wstack = jnp.stack([W_mi, W_me])
    z = _compute_z(xall, wstack)
    src = edge_index[0].astype(jnp.int32)
    dst = edge_index[1].astype(jnp.int32)
    e_arr, pmax = _edge_logits(z, src, dst)
    hp, pden = _aggregate(z[:, :FH], z[:, FH:], src, dst, e_arr, pmax)
    return _finalize(hp, pden)[:N]


# speedup vs baseline: 10.3099x; 1.1112x over previous
import functools

import jax
import jax.numpy as jnp
from jax import lax
from jax.experimental import pallas as pl
from jax.experimental.pallas import tpu as pltpu
from jax.experimental.pallas import tpu_sc as plsc

N_MI = 5000
N = 10000
D = 512
F = 128
E = 320000
SLOPE = 0.2

L = 16
NC = 2
NS = 16
NW = NC * NS
NPAD = 10240
EPT = E // NW
CH = 80
NCHUNK = EPT // CH
BPC = CH // L
CPT = NPAD // NS
NEG = -1e30


def _rot_perms():
    iota = lax.iota(jnp.int32, L)
    return [(iota + s) & (L - 1) for s in range(1, L)]


def _take(x, idx):
    return jnp.take_along_axis(x, idx, axis=0, mode="promise_in_bounds")


def _matmul_body(x_ref, w_ref, o_ref):
    o_ref[...] = jnp.dot(
        x_ref[...], w_ref[0],
        preferred_element_type=jnp.float32,
    )


def _compute_z(xall, wstack):
    BR = 1000
    return pl.pallas_call(
        _matmul_body,
        grid=(N // BR,),
        in_specs=[
            pl.BlockSpec((BR, D), lambda i: (i, 0)),
            pl.BlockSpec((1, D, F), lambda i: (i * BR // N_MI, 0, 0)),
        ],
        out_specs=pl.BlockSpec((BR, F), lambda i: (i, 0)),
        out_shape=jax.ShapeDtypeStruct((N, F), jnp.float32),
    )(xall, wstack)


def _edge_logits(z, srcv, dstv):
    mesh = plsc.VectorSubcoreMesh(core_axis_name="c", subcore_axis_name="s")

    @functools.partial(
        pl.kernel,
        out_type=(
            jax.ShapeDtypeStruct((E,), jnp.float32),
            jax.ShapeDtypeStruct((NW, NPAD), jnp.float32),
        ),
        mesh=mesh,
        scratch_types=[
            [pltpu.VMEM((CH,), jnp.int32)] * 2,
            [pltpu.VMEM((CH,), jnp.int32)] * 2,
            [pltpu.VMEM((CH, F), jnp.float32)] * 2,
            [pltpu.VMEM((CH, F), jnp.float32)] * 2,
            [pltpu.VMEM((CH,), jnp.float32)] * 2,
            pltpu.VMEM((CH,), jnp.int32),
            pltpu.VMEM((L, L), jnp.float32),
            pltpu.VMEM((NPAD,), jnp.float32),
            [pltpu.SemaphoreType.DMA] * 2,
            [pltpu.SemaphoreType.DMA] * 2,
            [pltpu.SemaphoreType.DMA] * 2,
        ],
        compiler_params=pltpu.CompilerParams(needs_layout_passes=False),
    )
    def k(z_hbm, src_hbm, dst_hbm, e_hbm, pmax_hbm,
          sidx, didx, zs, zd, ebuf, kbuf, scr, pmax_v, semI, semZ, semE):
        cid = lax.axis_index("c")
        sid = lax.axis_index("s")
        wid = sid * NC + cid
        base = wid * EPT
        perms = _rot_perms()
        iota = lax.iota(jnp.int32, L)

        def init_body(i, carry):
            pmax_v[pl.ds(i * L, L)] = jnp.full((L,), NEG, jnp.float32)
            return carry
        lax.fori_loop(0, NPAD // L, init_body, 0)

        def issue_idx(kc, b):
            off = base + kc * CH
            pltpu.async_copy(src_hbm.at[pl.ds(off, CH)], sidx[b], semI[b])
            pltpu.async_copy(dst_hbm.at[pl.ds(off, CH)], didx[b], semI[b])

        def wait_idx(b):
            pltpu.make_async_copy(src_hbm.at[pl.ds(0, CH)], sidx[b], semI[b]).wait()
            pltpu.make_async_copy(dst_hbm.at[pl.ds(0, CH)], didx[b], semI[b]).wait()

        def issue_gather(b):
            pltpu.async_copy(z_hbm.at[sidx[b]], zs[b], semZ[b])
            pltpu.async_copy(z_hbm.at[didx[b]], zd[b], semZ[b])

        def wait_gather(b):
            pltpu.make_async_copy(z_hbm.at[sidx[b]], zs[b], semZ[b]).wait()
            pltpu.make_async_copy(z_hbm.at[didx[b]], zd[b], semZ[b]).wait()

        def compute_chunk(kc, b):
            def block_body(j, bc):
                eb = j * L
                for t in range(L):
                    r = eb + t
                    acc = zs[b][r, pl.ds(0, L)] * zd[b][r, pl.ds(0, L)]
                    for c in range(1, F // L):
                        acc = acc + (zs[b][r, pl.ds(c * L, L)]
                                     * zd[b][r, pl.ds(c * L, L)])
                    scr[t] = acc
                dots = plsc.load_gather(scr, [iota, jnp.full((L,), 0, jnp.int32)])
                for l in range(1, L):
                    dots = dots + plsc.load_gather(
                        scr, [iota, jnp.full((L,), l, jnp.int32)])
                ev = jnp.where(dots > 0, dots, SLOPE * dots)
                ebuf[b][pl.ds(eb, L)] = ev
                key = kbuf[pl.ds(eb, L)]
                gmax = ev
                for perm in perms:
                    kr = _take(key, perm)
                    vr = _take(ev, perm)
                    gmax = jnp.maximum(gmax, jnp.where(kr == key, vr, NEG))
                old = plsc.load_gather(pmax_v, [key])
                plsc.store_scatter(pmax_v, [key], jnp.maximum(old, gmax))
                return bc
            lax.fori_loop(0, BPC, block_body, 0)
            pltpu.async_copy(ebuf[b], e_hbm.at[pl.ds(base + kc * CH, CH)], semE[b])

        def process(kc, b, wait_e, next_gather, idx2):
            wait_gather(b)
            if next_gather:
                wait_idx(1 - b)
                issue_gather(1 - b)
            for v in range(CH // L):
                kbuf[pl.ds(v * L, L)] = didx[b][pl.ds(v * L, L)]
            if idx2 == "always":
                issue_idx(kc + 2, b)
            elif idx2 == "guard":
                @pl.when(kc + 2 < NCHUNK)
                def _():
                    issue_idx(kc + 2, b)
            if wait_e:
                pltpu.make_async_copy(
                    ebuf[b], e_hbm.at[pl.ds(0, CH)], semE[b]).wait()
            compute_chunk(kc, b)

        pltpu.sync_copy(src_hbm.at[pl.ds(base, CH)], sidx[0])
        pltpu.sync_copy(dst_hbm.at[pl.ds(base, CH)], didx[0])
        issue_gather(0)
        issue_idx(1, 1)
        process(0, 0, False, True, "always")
        process(1, 1, False, True, "always")

        @pl.loop(2, NCHUNK - 1, step=2)
        def _(kc):
            process(kc, 0, True, True, "always")
            process(kc + 1, 1, True, True, "guard")

        process(NCHUNK - 1, 0, True, False, "skip")
        pltpu.make_async_copy(ebuf[0], e_hbm.at[pl.ds(0, CH)], semE[0]).wait()
        pltpu.make_async_copy(ebuf[1], e_hbm.at[pl.ds(0, CH)], semE[1]).wait()
        pltpu.sync_copy(pmax_v, pmax_hbm.at[wid])

    return k(z, srcv, dstv)


FH = F // 2


def _aggregate(z0, z1, srcv, dstv, e_arr, pmax):
    mesh = plsc.VectorSubcoreMesh(core_axis_name="c", subcore_axis_name="s")

    @functools.partial(
        pl.kernel,
        out_type=(
            jax.ShapeDtypeStruct((NC, 2, NPAD, FH), jnp.float32),
            jax.ShapeDtypeStruct((NW, NPAD), jnp.float32),
        ),
        mesh=mesh,
        scratch_types=[
            pltpu.VMEM((NW, CPT), jnp.float32),
            pltpu.VMEM((CPT,), jnp.float32),
            pltpu.VMEM((NPAD,), jnp.float32),
            pltpu.VMEM((NPAD,), jnp.float32),
            pltpu.VMEM((EPT,), jnp.float32),
            [pltpu.VMEM((CH,), jnp.int32)] * 2,
            [pltpu.VMEM((CH,), jnp.int32)] * 2,
            [pltpu.VMEM((CH,), jnp.int32)] * 2,
            [pltpu.VMEM((CH,), jnp.float32)] * 2,
            [pltpu.VMEM((CH, FH), jnp.float32)] * 2,
            pltpu.VMEM((CH, FH), jnp.float32),
            pltpu.VMEM_SHARED((NPAD,), jnp.float32),
            pltpu.VMEM_SHARED((NPAD, FH), jnp.float32),
            [pltpu.SemaphoreType.DMA] * 2,
            [pltpu.SemaphoreType.DMA] * 2,
            [pltpu.SemaphoreType.DMA] * 2,
        ],
        compiler_params=pltpu.CompilerParams(
            needs_layout_passes=False, use_tc_tiling_on_sc=False),
    )
    def k(z0_hbm, z1_hbm, src_hbm, dst_hbm, e_hbm, pmax_hbm, hp_hbm, pden_hbm,
          mtmp, msl, mx_full, pden_v, exal, sidx, didx, sdidx, ebuf, zs, zbuf,
          mx_sh, h_sh, semI, semZ, semA):
        cid = lax.axis_index("c")
        sid = lax.axis_index("s")
        wid = sid * NC + cid
        base = wid * EPT
        col0 = sid * CPT
        perms = _rot_perms()

        for r in range(NW):
            pltpu.sync_copy(pmax_hbm.at[r, pl.ds(col0, CPT)], mtmp.at[r])

        def mred(v, carry):
            m = mtmp[0, pl.ds(v * L, L)]
            for r in range(1, NW):
                m = jnp.maximum(m, mtmp[r, pl.ds(v * L, L)])
            msl[pl.ds(v * L, L)] = m
            return carry
        lax.fori_loop(0, CPT // L, mred, 0)
        pltpu.sync_copy(msl, mx_sh.at[pl.ds(col0, CPT)])

        def zrow(i, carry):
            for c in range(FH // L):
                zbuf[i, pl.ds(c * L, L)] = jnp.zeros((L,), jnp.float32)
            return carry
        lax.fori_loop(0, CH, zrow, 0)

        def zden(i, carry):
            pden_v[pl.ds(i * L, L)] = jnp.zeros((L,), jnp.float32)
            return carry
        lax.fori_loop(0, NPAD // L, zden, 0)

        plsc.subcore_barrier()
        pltpu.sync_copy(mx_sh, mx_full)

        for half in range(2):
            zh_hbm = z0_hbm if half == 0 else z1_hbm
            for b in range(CPT // CH):
                pltpu.sync_copy(zbuf, h_sh.at[pl.ds(col0 + b * CH, CH)])
            plsc.subcore_barrier()

            def issue_idx(kc, b):
                off = base + kc * CH
                pltpu.async_copy(src_hbm.at[pl.ds(off, CH)], sidx[b], semI[b])
                pltpu.async_copy(dst_hbm.at[pl.ds(off, CH)], didx[b], semI[b])

            def issue_e(kc, b):
                if half == 0:
                    off = base + kc * CH
                    pltpu.async_copy(e_hbm.at[pl.ds(off, CH)], ebuf[b], semI[b])

            def wait_idx(b):
                pltpu.make_async_copy(
                    src_hbm.at[pl.ds(0, CH)], sidx[b], semI[b]).wait()
                pltpu.make_async_copy(
                    dst_hbm.at[pl.ds(0, CH)], didx[b], semI[b]).wait()
                if half == 0:
                    pltpu.make_async_copy(
                        e_hbm.at[pl.ds(0, CH)], ebuf[b], semI[b]).wait()

            def issue_gather(b):
                pltpu.async_copy(zh_hbm.at[sidx[b]], zs[b], semZ[b])

            def wait_gather(b):
                pltpu.make_async_copy(zh_hbm.at[sidx[b]], zs[b], semZ[b]).wait()

            def drain_scatter(b):
                pltpu.make_async_copy(
                    zs[b], h_sh.at[sdidx[b]], semA[b]).wait()

            def compute_chunk(kc, b):
                loc = kc * CH

                def block_body(j, bc):
                    eb = j * L
                    key = sdidx[b][pl.ds(eb, L)]
                    if half == 0:
                        ev = ebuf[b][pl.ds(eb, L)]
                        mx = plsc.load_gather(mx_full, [key])
                        ex = jnp.exp(ev - mx)
                        exal[pl.ds(loc + eb, L)] = ex
                        tot = ex
                        for perm in perms:
                            kr = _take(key, perm)
                            vr = _take(ex, perm)
                            tot = tot + jnp.where(kr == key, vr, 0.0)
                        old = plsc.load_gather(pden_v, [key])
                        plsc.store_scatter(pden_v, [key], old + tot)
                    else:
                        ex = exal[pl.ds(loc + eb, L)]
                    for t in range(L):
                        r = eb + t
                        al = _take(ex, jnp.full((L,), t, jnp.int32))
                        for c in range(FH // L):
                            zs[b][r, pl.ds(c * L, L)] = (
                                zs[b][r, pl.ds(c * L, L)] * al)
                    return bc
                lax.fori_loop(0, BPC, block_body, 0)
                pltpu.async_copy(zs[b], h_sh.at[sdidx[b]], semA[b], add=True)

            def process(kc, b, next_gather, drain_a, idx2):
                wait_gather(b)
                if next_gather:
                    if drain_a:
                        drain_scatter(1 - b)
                    wait_idx(1 - b)
                    issue_gather(1 - b)
                for v in range(CH // L):
                    sdidx[b][pl.ds(v * L, L)] = didx[b][pl.ds(v * L, L)]
                if idx2 == "always":
                    issue_idx(kc + 2, b)
                elif idx2 == "guard":
                    @pl.when(kc + 2 < NCHUNK)
                    def _():
                        issue_idx(kc + 2, b)
                compute_chunk(kc, b)
                if idx2 == "always":
                    issue_e(kc + 2, b)
                elif idx2 == "guard":
                    @pl.when(kc + 2 < NCHUNK)
                    def _():
                        issue_e(kc + 2, b)

            pltpu.sync_copy(src_hbm.at[pl.ds(base, CH)], sidx[0])
            pltpu.sync_copy(dst_hbm.at[pl.ds(base, CH)], didx[0])
            if half == 0:
                pltpu.sync_copy(e_hbm.at[pl.ds(base, CH)], ebuf[0])
            issue_gather(0)
            issue_idx(1, 1)
            issue_e(1, 1)
            process(0, 0, True, False, "always")
            process(1, 1, True, True, "always")

            @pl.loop(2, NCHUNK - 1, step=2)
            def _(kc):
                process(kc, 0, True, True, "always")
                process(kc + 1, 1, True, True, "guard")

            process(NCHUNK - 1, 0, False, False, "skip")
            drain_scatter(0)
            drain_scatter(1)

            plsc.subcore_barrier()
            pltpu.sync_copy(h_sh.at[pl.ds(col0, CPT)],
                            hp_hbm.at[cid, half, pl.ds(col0, CPT)])
            plsc.subcore_barrier()

        pltpu.sync_copy(pden_v, pden_hbm.at[wid])

    return k(z0, z1, srcv, dstv, e_arr, pmax)


def _finalize(hp, pden):
    BR = 2048

    def body(h_ref, d_ref, o_ref):
        den = jnp.sum(d_ref[...], axis=0)[:, None] + 1e-9
        for half in range(2):
            val = (h_ref[0, half] + h_ref[1, half]) / den
            o_ref[:, half * FH:(half + 1) * FH] = jnp.where(
                val > 0, val, jnp.exp(val) - 1.0)

    return pl.pallas_call(
        body,
        grid=(NPAD // BR,),
        in_specs=[
            pl.BlockSpec((NC, 2, BR, FH), lambda i: (0, 0, i, 0)),
            pl.BlockSpec((NW, BR), lambda i: (0, i)),
        ],
        out_specs=pl.BlockSpec((BR, F), lambda i: (i, 0)),
        out_shape=jax.ShapeDtypeStruct((NPAD, F), jnp.float32),
    )(hp, pden)


def kernel(mi_sim, me_sim, edge_index, W_mi, W_me):
    xall = jnp.concatenate([mi_sim, me_sim], axis=0)
    wstack = jnp.stack([W_mi, W_me])
    z = _compute_z(xall, wstack)
    src = edge_index[0].astype(jnp.int32)
    dst = edge_index[1].astype(jnp.int32)
    e_arr, pmax = _edge_logits(z, src, dst)
    hp, pden = _aggregate(z[:, :FH], z[:, FH:], src, dst, e_arr, pmax)
    return _finalize(hp, pden)[:N]


# speedup vs baseline: 10.5741x; 1.0256x over previous
import functools

import jax
import jax.numpy as jnp
from jax import lax
from jax.experimental import pallas as pl
from jax.experimental.pallas import tpu as pltpu
from jax.experimental.pallas import tpu_sc as plsc

N_MI = 5000
N = 10000
D = 512
F = 128
E = 320000
SLOPE = 0.2

L = 16
NC = 2
NS = 16
NW = NC * NS
NPAD = 10240
EPT = E // NW
CH = 80
NCHUNK = EPT // CH
BPC = CH // L
CPT = NPAD // NS
NEG = -1e30


def _rot_perms():
    iota = lax.iota(jnp.int32, L)
    return [(iota + s) & (L - 1) for s in range(1, L)]


def _take(x, idx):
    return jnp.take_along_axis(x, idx, axis=0, mode="promise_in_bounds")


def _matmul_body(x_ref, w_ref, o_ref):
    o_ref[...] = jnp.dot(
        x_ref[...], w_ref[0],
        preferred_element_type=jnp.float32,
    )


def _compute_z(xall, wstack):
    BR = 1000
    return pl.pallas_call(
        _matmul_body,
        grid=(N // BR,),
        in_specs=[
            pl.BlockSpec((BR, D), lambda i: (i, 0)),
            pl.BlockSpec((1, D, F), lambda i: (i * BR // N_MI, 0, 0)),
        ],
        out_specs=pl.BlockSpec((BR, F), lambda i: (i, 0)),
        out_shape=jax.ShapeDtypeStruct((N, F), jnp.float32),
    )(xall, wstack)


def _edge_logits(z, srcv, dstv):
    mesh = plsc.VectorSubcoreMesh(core_axis_name="c", subcore_axis_name="s")

    @functools.partial(
        pl.kernel,
        out_type=(
            jax.ShapeDtypeStruct((E,), jnp.float32),
            jax.ShapeDtypeStruct((NW, NPAD), jnp.float32),
        ),
        mesh=mesh,
        scratch_types=[
            pltpu.VMEM((EPT,), jnp.int32),
            pltpu.VMEM((EPT,), jnp.int32),
            [pltpu.VMEM((CH, F), jnp.float32)] * 2,
            [pltpu.VMEM((CH, F), jnp.float32)] * 2,
            [pltpu.VMEM((CH,), jnp.float32)] * 2,
            pltpu.VMEM((L, L), jnp.float32),
            pltpu.VMEM((NPAD,), jnp.float32),
            [pltpu.SemaphoreType.DMA] * 2,
            [pltpu.SemaphoreType.DMA] * 2,
        ],
        compiler_params=pltpu.CompilerParams(needs_layout_passes=False),
    )
    def k(z_hbm, src_hbm, dst_hbm, e_hbm, pmax_hbm,
          src_all, dst_all, zs, zd, ebuf, scr, pmax_v, semZ, semE):
        cid = lax.axis_index("c")
        sid = lax.axis_index("s")
        wid = sid * NC + cid
        base = wid * EPT
        perms = _rot_perms()
        iota = lax.iota(jnp.int32, L)

        pltpu.sync_copy(src_hbm.at[pl.ds(base, EPT)], src_all)
        pltpu.sync_copy(dst_hbm.at[pl.ds(base, EPT)], dst_all)

        def init_body(i, carry):
            pmax_v[pl.ds(i * L, L)] = jnp.full((L,), NEG, jnp.float32)
            return carry
        lax.fori_loop(0, NPAD // L, init_body, 0)

        def issue_gather(kc, b):
            loc = kc * CH
            pltpu.async_copy(z_hbm.at[src_all.at[pl.ds(loc, CH)]], zs[b], semZ[b])
            pltpu.async_copy(z_hbm.at[dst_all.at[pl.ds(loc, CH)]], zd[b], semZ[b])

        def wait_gather(kc, b):
            loc = kc * CH
            pltpu.make_async_copy(
                z_hbm.at[src_all.at[pl.ds(loc, CH)]], zs[b], semZ[b]).wait()
            pltpu.make_async_copy(
                z_hbm.at[dst_all.at[pl.ds(loc, CH)]], zd[b], semZ[b]).wait()

        def compute_chunk(kc, b):
            loc = kc * CH

            def block_body(j, bc):
                eb = j * L
                for t in range(L):
                    r = eb + t
                    acc = zs[b][r, pl.ds(0, L)] * zd[b][r, pl.ds(0, L)]
                    for c in range(1, F // L):
                        acc = acc + (zs[b][r, pl.ds(c * L, L)]
                                     * zd[b][r, pl.ds(c * L, L)])
                    scr[t] = acc
                dots = plsc.load_gather(scr, [iota, jnp.full((L,), 0, jnp.int32)])
                for l in range(1, L):
                    dots = dots + plsc.load_gather(
                        scr, [iota, jnp.full((L,), l, jnp.int32)])
                ev = jnp.where(dots > 0, dots, SLOPE * dots)
                ebuf[b][pl.ds(eb, L)] = ev
                key = dst_all[pl.ds(loc + eb, L)]
                gmax = ev
                for perm in perms:
                    kr = _take(key, perm)
                    vr = _take(ev, perm)
                    gmax = jnp.maximum(gmax, jnp.where(kr == key, vr, NEG))
                old = plsc.load_gather(pmax_v, [key])
                plsc.store_scatter(pmax_v, [key], jnp.maximum(old, gmax))
                return bc
            lax.fori_loop(0, BPC, block_body, 0)
            pltpu.async_copy(ebuf[b], e_hbm.at[pl.ds(base + kc * CH, CH)], semE[b])

        def process(kc, b, wait_e, next_gather):
            wait_gather(kc, b)
            if next_gather:
                issue_gather(kc + 1, 1 - b)
            if wait_e:
                pltpu.make_async_copy(
                    ebuf[b], e_hbm.at[pl.ds(0, CH)], semE[b]).wait()
            compute_chunk(kc, b)

        issue_gather(0, 0)
        process(0, 0, False, True)
        process(1, 1, False, True)

        @pl.loop(2, NCHUNK - 1, step=2)
        def _(kc):
            process(kc, 0, True, True)
            process(kc + 1, 1, True, True)

        process(NCHUNK - 1, 0, True, False)
        pltpu.make_async_copy(ebuf[0], e_hbm.at[pl.ds(0, CH)], semE[0]).wait()
        pltpu.make_async_copy(ebuf[1], e_hbm.at[pl.ds(0, CH)], semE[1]).wait()
        pltpu.sync_copy(pmax_v, pmax_hbm.at[wid])

    return k(z, srcv, dstv)


FH = F // 2


def _aggregate(z0, z1, srcv, dstv, e_arr, pmax):
    mesh = plsc.VectorSubcoreMesh(core_axis_name="c", subcore_axis_name="s")

    @functools.partial(
        pl.kernel,
        out_type=(
            jax.ShapeDtypeStruct((NC, 2, NPAD, FH), jnp.float32),
            jax.ShapeDtypeStruct((NW, NPAD), jnp.float32),
        ),
        mesh=mesh,
        scratch_types=[
            pltpu.VMEM((4, CPT), jnp.float32),
            pltpu.VMEM((CPT,), jnp.float32),
            pltpu.VMEM((NPAD,), jnp.float32),
            pltpu.VMEM((NPAD,), jnp.float32),
            pltpu.VMEM((EPT,), jnp.float32),
            pltpu.VMEM((EPT,), jnp.int32),
            pltpu.VMEM((EPT,), jnp.int32),
            [pltpu.VMEM((CH,), jnp.int32)] * 2,
            [pltpu.VMEM((CH, FH), jnp.float32)] * 2,
            pltpu.VMEM((CH, FH), jnp.float32),
            pltpu.VMEM_SHARED((NPAD,), jnp.float32),
            pltpu.VMEM_SHARED((NPAD, FH), jnp.float32),
            [pltpu.SemaphoreType.DMA] * 2,
            [pltpu.SemaphoreType.DMA] * 2,
        ],
        compiler_params=pltpu.CompilerParams(
            needs_layout_passes=False, use_tc_tiling_on_sc=False),
    )
    def k(z0_hbm, z1_hbm, src_hbm, dst_hbm, e_hbm, pmax_hbm, hp_hbm, pden_hbm,
          mtmp, msl, mx_full, pden_v, exal, src_all, dst_all, sdidx,
          zs, zbuf, mx_sh, h_sh, semZ, semA):
        cid = lax.axis_index("c")
        sid = lax.axis_index("s")
        wid = sid * NC + cid
        base = wid * EPT
        col0 = sid * CPT
        perms = _rot_perms()

        pltpu.sync_copy(src_hbm.at[pl.ds(base, EPT)], src_all)
        pltpu.sync_copy(dst_hbm.at[pl.ds(base, EPT)], dst_all)
        pltpu.sync_copy(e_hbm.at[pl.ds(base, EPT)], exal)

        for r0 in range(0, NW, 4):
            for r in range(4):
                pltpu.sync_copy(pmax_hbm.at[r0 + r, pl.ds(col0, CPT)],
                                mtmp.at[r])

            def mred(v, carry):
                m = mtmp[0, pl.ds(v * L, L)]
                for r in range(1, 4):
                    m = jnp.maximum(m, mtmp[r, pl.ds(v * L, L)])
                if r0 > 0:
                    m = jnp.maximum(m, msl[pl.ds(v * L, L)])
                msl[pl.ds(v * L, L)] = m
                return carry
            lax.fori_loop(0, CPT // L, mred, 0)
        pltpu.sync_copy(msl, mx_sh.at[pl.ds(col0, CPT)])

        def zrow(i, carry):
            for c in range(FH // L):
                zbuf[i, pl.ds(c * L, L)] = jnp.zeros((L,), jnp.float32)
            return carry
        lax.fori_loop(0, CH, zrow, 0)

        def zden(i, carry):
            pden_v[pl.ds(i * L, L)] = jnp.zeros((L,), jnp.float32)
            return carry
        lax.fori_loop(0, NPAD // L, zden, 0)

        plsc.subcore_barrier()
        pltpu.sync_copy(mx_sh, mx_full)

        for half in range(2):
            zh_hbm = z0_hbm if half == 0 else z1_hbm
            for b in range(CPT // CH):
                pltpu.sync_copy(zbuf, h_sh.at[pl.ds(col0 + b * CH, CH)])
            plsc.subcore_barrier()

            def issue_gather(kc, b):
                loc = kc * CH
                pltpu.async_copy(
                    zh_hbm.at[src_all.at[pl.ds(loc, CH)]], zs[b], semZ[b])

            def wait_gather(kc, b):
                loc = kc * CH
                pltpu.make_async_copy(
                    zh_hbm.at[src_all.at[pl.ds(loc, CH)]], zs[b], semZ[b]).wait()

            def drain_scatter(b):
                pltpu.make_async_copy(
                    zs[b], h_sh.at[sdidx[b]], semA[b]).wait()

            def compute_chunk(kc, b):
                loc = kc * CH
                for v in range(CH // L):
                    sdidx[b][pl.ds(v * L, L)] = dst_all[pl.ds(loc + v * L, L)]

                def block_body(j, bc):
                    eb = j * L
                    key = dst_all[pl.ds(loc + eb, L)]
                    if half == 0:
                        ev = exal[pl.ds(loc + eb, L)]
                        mx = plsc.load_gather(mx_full, [key])
                        ex = jnp.exp(ev - mx)
                        exal[pl.ds(loc + eb, L)] = ex
                        tot = ex
                        for perm in perms:
                            kr = _take(key, perm)
                            vr = _take(ex, perm)
                            tot = tot + jnp.where(kr == key, vr, 0.0)
                        old = plsc.load_gather(pden_v, [key])
                        plsc.store_scatter(pden_v, [key], old + tot)
                    else:
                        ex = exal[pl.ds(loc + eb, L)]
                    for t in range(L):
                        r = eb + t
                        al = _take(ex, jnp.full((L,), t, jnp.int32))
                        for c in range(FH // L):
                            zs[b][r, pl.ds(c * L, L)] = (
                                zs[b][r, pl.ds(c * L, L)] * al)
                    return bc
                lax.fori_loop(0, BPC, block_body, 0)
                pltpu.async_copy(zs[b], h_sh.at[sdidx[b]], semA[b], add=True)

            def process(kc, b, next_gather, drain_a):
                wait_gather(kc, b)
                if next_gather:
                    if drain_a:
                        drain_scatter(1 - b)
                    issue_gather(kc + 1, 1 - b)
                compute_chunk(kc, b)

            issue_gather(0, 0)
            process(0, 0, True, False)
            process(1, 1, True, True)

            @pl.loop(2, NCHUNK - 1, step=2)
            def _(kc):
                process(kc, 0, True, True)
                process(kc + 1, 1, True, True)

            process(NCHUNK - 1, 0, False, False)
            drain_scatter(0)
            drain_scatter(1)

            plsc.subcore_barrier()
            pltpu.sync_copy(h_sh.at[pl.ds(col0, CPT)],
                            hp_hbm.at[cid, half, pl.ds(col0, CPT)])
            plsc.subcore_barrier()

        pltpu.sync_copy(pden_v, pden_hbm.at[wid])

    return k(z0, z1, srcv, dstv, e_arr, pmax)


def _finalize(hp, pden):
    BR = 2048

    def body(h_ref, d_ref, o_ref):
        den = jnp.sum(d_ref[...], axis=0)[:, None] + 1e-9
        for half in range(2):
            val = (h_ref[0, half] + h_ref[1, half]) / den
            o_ref[:, half * FH:(half + 1) * FH] = jnp.where(
                val > 0, val, jnp.exp(val) - 1.0)

    return pl.pallas_call(
        body,
        grid=(NPAD // BR,),
        in_specs=[
            pl.BlockSpec((NC, 2, BR, FH), lambda i: (0, 0, i, 0)),
            pl.BlockSpec((NW, BR), lambda i: (0, i)),
        ],
        out_specs=pl.BlockSpec((BR, F), lambda i: (i, 0)),
        out_shape=jax.ShapeDtypeStruct((NPAD, F), jnp.float32),
    )(hp, pden)


def kernel(mi_sim, me_sim, edge_index, W_mi, W_me):
    xall = jnp.concatenate([mi_sim, me_sim], axis=0)
    wstack = jnp.stack([W_mi, W_me])
    z = _compute_z(xall, wstack)
    src = edge_index[0].astype(jnp.int32)
    dst = edge_index[1].astype(jnp.int32)
    e_arr, pmax = _edge_logits(z, src, dst)
    hp, pden = _aggregate(z[:, :FH], z[:, FH:], src, dst, e_arr, pmax)
    return _finalize(hp, pden)[:N]


# speedup vs baseline: 10.5935x; 1.0018x over previous
import functools

import jax
import jax.numpy as jnp
from jax import lax
from jax.experimental import pallas as pl
from jax.experimental.pallas import tpu as pltpu
from jax.experimental.pallas import tpu_sc as plsc

N_MI = 5000
N = 10000
D = 512
F = 128
E = 320000
SLOPE = 0.2

L = 16
NC = 2
NS = 16
NW = NC * NS
NPAD = 10240
EPT = E // NW
CH = 80
NCHUNK = EPT // CH
BPC = CH // L
CPT = NPAD // NS
NEG = -1e30


def _rot_perms():
    iota = lax.iota(jnp.int32, L)
    return [(iota + s) & (L - 1) for s in range(1, L)]


def _take(x, idx):
    return jnp.take_along_axis(x, idx, axis=0, mode="promise_in_bounds")


def _matmul_body(x_ref, w_ref, o_ref):
    o_ref[...] = jnp.dot(
        x_ref[...], w_ref[0],
        preferred_element_type=jnp.float32,
    )


def _compute_z(xall, wstack):
    BR = 1000
    return pl.pallas_call(
        _matmul_body,
        grid=(N // BR,),
        in_specs=[
            pl.BlockSpec((BR, D), lambda i: (i, 0)),
            pl.BlockSpec((1, D, F), lambda i: (i * BR // N_MI, 0, 0)),
        ],
        out_specs=pl.BlockSpec((BR, F), lambda i: (i, 0)),
        out_shape=jax.ShapeDtypeStruct((N, F), jnp.float32),
    )(xall, wstack)


def _edge_logits(z, srcv, dstv):
    mesh = plsc.VectorSubcoreMesh(core_axis_name="c", subcore_axis_name="s")

    @functools.partial(
        pl.kernel,
        out_type=(
            jax.ShapeDtypeStruct((E,), jnp.float32),
            jax.ShapeDtypeStruct((NW, NPAD), jnp.float32),
        ),
        mesh=mesh,
        scratch_types=[
            pltpu.VMEM((EPT,), jnp.int32),
            pltpu.VMEM((EPT,), jnp.int32),
            [pltpu.VMEM((CH, F), jnp.float32)] * 2,
            [pltpu.VMEM((CH, F), jnp.float32)] * 2,
            [pltpu.VMEM((CH,), jnp.float32)] * 2,
            pltpu.VMEM((L, L), jnp.float32),
            pltpu.VMEM((NPAD,), jnp.float32),
            [pltpu.SemaphoreType.DMA] * 2,
            [pltpu.SemaphoreType.DMA] * 2,
        ],
        compiler_params=pltpu.CompilerParams(needs_layout_passes=False),
    )
    def k(z_hbm, src_hbm, dst_hbm, e_hbm, pmax_hbm,
          src_all, dst_all, zs, zd, ebuf, scr, pmax_v, semZ, semE):
        cid = lax.axis_index("c")
        sid = lax.axis_index("s")
        wid = sid * NC + cid
        base = wid * EPT
        perms = _rot_perms()
        iota = lax.iota(jnp.int32, L)

        pltpu.sync_copy(src_hbm.at[pl.ds(base, EPT)], src_all)
        pltpu.sync_copy(dst_hbm.at[pl.ds(base, EPT)], dst_all)

        def init_body(i, carry):
            pmax_v[pl.ds(i * L, L)] = jnp.full((L,), NEG, jnp.float32)
            return carry
        lax.fori_loop(0, NPAD // L, init_body, 0)

        def issue_gather(kc, b):
            loc = kc * CH
            pltpu.async_copy(z_hbm.at[src_all.at[pl.ds(loc, CH)]], zs[b], semZ[b])
            pltpu.async_copy(z_hbm.at[dst_all.at[pl.ds(loc, CH)]], zd[b], semZ[b])

        def wait_gather(kc, b):
            loc = kc * CH
            pltpu.make_async_copy(
                z_hbm.at[src_all.at[pl.ds(loc, CH)]], zs[b], semZ[b]).wait()
            pltpu.make_async_copy(
                z_hbm.at[dst_all.at[pl.ds(loc, CH)]], zd[b], semZ[b]).wait()

        def compute_chunk(kc, b):
            loc = kc * CH

            def block_body(j, bc):
                eb = j * L
                for t in range(L):
                    r = eb + t
                    acc = zs[b][r, pl.ds(0, L)] * zd[b][r, pl.ds(0, L)]
                    for c in range(1, F // L):
                        acc = acc + (zs[b][r, pl.ds(c * L, L)]
                                     * zd[b][r, pl.ds(c * L, L)])
                    scr[t] = acc
                dots = plsc.load_gather(scr, [iota, jnp.full((L,), 0, jnp.int32)])
                for l in range(1, L):
                    dots = dots + plsc.load_gather(
                        scr, [iota, jnp.full((L,), l, jnp.int32)])
                ev = jnp.where(dots > 0, dots, SLOPE * dots)
                ebuf[b][pl.ds(eb, L)] = ev
                key = dst_all[pl.ds(loc + eb, L)]
                gmax = ev
                for perm in perms:
                    kr = _take(key, perm)
                    vr = _take(ev, perm)
                    gmax = jnp.maximum(gmax, jnp.where(kr == key, vr, NEG))
                old = plsc.load_gather(pmax_v, [key])
                plsc.store_scatter(pmax_v, [key], jnp.maximum(old, gmax))
                return bc
            lax.fori_loop(0, BPC, block_body, 0)
            pltpu.async_copy(ebuf[b], e_hbm.at[pl.ds(base + kc * CH, CH)], semE[b])

        def process(kc, b, wait_e, next_gather):
            wait_gather(kc, b)
            if next_gather:
                issue_gather(kc + 1, 1 - b)
            if wait_e:
                pltpu.make_async_copy(
                    ebuf[b], e_hbm.at[pl.ds(0, CH)], semE[b]).wait()
            compute_chunk(kc, b)

        issue_gather(0, 0)
        process(0, 0, False, True)
        process(1, 1, False, True)

        @pl.loop(2, NCHUNK - 1, step=2)
        def _(kc):
            process(kc, 0, True, True)
            process(kc + 1, 1, True, True)

        process(NCHUNK - 1, 0, True, False)
        pltpu.make_async_copy(ebuf[0], e_hbm.at[pl.ds(0, CH)], semE[0]).wait()
        pltpu.make_async_copy(ebuf[1], e_hbm.at[pl.ds(0, CH)], semE[1]).wait()
        pltpu.sync_copy(pmax_v, pmax_hbm.at[wid])

    return k(z, srcv, dstv)


FH = F // 2


def _aggregate(z0, z1, srcv, dstv, e_arr, pmax):
    mesh = plsc.VectorSubcoreMesh(core_axis_name="c", subcore_axis_name="s")

    @functools.partial(
        pl.kernel,
        out_type=(
            jax.ShapeDtypeStruct((NC, 2, NPAD, FH), jnp.float32),
            jax.ShapeDtypeStruct((NW, NPAD), jnp.float32),
        ),
        mesh=mesh,
        scratch_types=[
            pltpu.VMEM((4, CPT), jnp.float32),
            pltpu.VMEM((CPT,), jnp.float32),
            pltpu.VMEM((NPAD,), jnp.float32),
            pltpu.VMEM((NPAD,), jnp.float32),
            pltpu.VMEM((EPT,), jnp.float32),
            pltpu.VMEM((EPT,), jnp.int32),
            pltpu.VMEM((EPT,), jnp.int32),
            [pltpu.VMEM((CH,), jnp.int32)] * 4,
            [pltpu.VMEM((CH, FH), jnp.float32)] * 4,
            pltpu.VMEM((CH, FH), jnp.float32),
            pltpu.VMEM_SHARED((NPAD,), jnp.float32),
            pltpu.VMEM_SHARED((NPAD, FH), jnp.float32),
            [pltpu.SemaphoreType.DMA] * 4,
            [pltpu.SemaphoreType.DMA] * 4,
        ],
        compiler_params=pltpu.CompilerParams(
            needs_layout_passes=False, use_tc_tiling_on_sc=False),
    )
    def k(z0_hbm, z1_hbm, src_hbm, dst_hbm, e_hbm, pmax_hbm, hp_hbm, pden_hbm,
          mtmp, msl, mx_full, pden_v, exal, src_all, dst_all, sdidx,
          zs, zbuf, mx_sh, h_sh, semZ, semA):
        cid = lax.axis_index("c")
        sid = lax.axis_index("s")
        wid = sid * NC + cid
        base = wid * EPT
        col0 = sid * CPT
        perms = _rot_perms()

        pltpu.sync_copy(src_hbm.at[pl.ds(base, EPT)], src_all)
        pltpu.sync_copy(dst_hbm.at[pl.ds(base, EPT)], dst_all)
        pltpu.sync_copy(e_hbm.at[pl.ds(base, EPT)], exal)

        for r0 in range(0, NW, 4):
            for r in range(4):
                pltpu.sync_copy(pmax_hbm.at[r0 + r, pl.ds(col0, CPT)],
                                mtmp.at[r])

            def mred(v, carry):
                m = mtmp[0, pl.ds(v * L, L)]
                for r in range(1, 4):
                    m = jnp.maximum(m, mtmp[r, pl.ds(v * L, L)])
                if r0 > 0:
                    m = jnp.maximum(m, msl[pl.ds(v * L, L)])
                msl[pl.ds(v * L, L)] = m
                return carry
            lax.fori_loop(0, CPT // L, mred, 0)
        pltpu.sync_copy(msl, mx_sh.at[pl.ds(col0, CPT)])

        def zrow(i, carry):
            for c in range(FH // L):
                zbuf[i, pl.ds(c * L, L)] = jnp.zeros((L,), jnp.float32)
            return carry
        lax.fori_loop(0, CH, zrow, 0)

        def zden(i, carry):
            pden_v[pl.ds(i * L, L)] = jnp.zeros((L,), jnp.float32)
            return carry
        lax.fori_loop(0, NPAD // L, zden, 0)

        plsc.subcore_barrier()
        pltpu.sync_copy(mx_sh, mx_full)

        for half in range(2):
            zh_hbm = z0_hbm if half == 0 else z1_hbm
            for b in range(CPT // CH):
                pltpu.sync_copy(zbuf, h_sh.at[pl.ds(col0 + b * CH, CH)])
            plsc.subcore_barrier()

            def issue_gather(kc, b):
                loc = kc * CH
                pltpu.async_copy(
                    zh_hbm.at[src_all.at[pl.ds(loc, CH)]], zs[b], semZ[b])

            def wait_gather(kc, b):
                loc = kc * CH
                pltpu.make_async_copy(
                    zh_hbm.at[src_all.at[pl.ds(loc, CH)]], zs[b], semZ[b]).wait()

            def drain_scatter(b):
                pltpu.make_async_copy(
                    zs[b], h_sh.at[sdidx[b]], semA[b]).wait()

            def compute_chunk(kc, b):
                loc = kc * CH
                for v in range(CH // L):
                    sdidx[b][pl.ds(v * L, L)] = dst_all[pl.ds(loc + v * L, L)]

                def block_body(j, bc):
                    eb = j * L
                    key = dst_all[pl.ds(loc + eb, L)]
                    if half == 0:
                        ev = exal[pl.ds(loc + eb, L)]
                        mx = plsc.load_gather(mx_full, [key])
                        ex = jnp.exp(ev - mx)
                        exal[pl.ds(loc + eb, L)] = ex
                        tot = ex
                        for perm in perms:
                            kr = _take(key, perm)
                            vr = _take(ex, perm)
                            tot = tot + jnp.where(kr == key, vr, 0.0)
                        old = plsc.load_gather(pden_v, [key])
                        plsc.store_scatter(pden_v, [key], old + tot)
                    else:
                        ex = exal[pl.ds(loc + eb, L)]
                    for t in range(L):
                        r = eb + t
                        al = _take(ex, jnp.full((L,), t, jnp.int32))
                        for c in range(FH // L):
                            zs[b][r, pl.ds(c * L, L)] = (
                                zs[b][r, pl.ds(c * L, L)] * al)
                    return bc
                lax.fori_loop(0, BPC, block_body, 0)
                pltpu.async_copy(zs[b], h_sh.at[sdidx[b]], semA[b], add=True)

            def process(kc, b, next_gather, drain_a, guard_next=False):
                wait_gather(kc, b)
                nb = (b + 1) % 4
                if next_gather:
                    if drain_a:
                        drain_scatter(nb)
                    if guard_next:
                        @pl.when(kc + 1 < NCHUNK)
                        def _():
                            issue_gather(kc + 1, nb)
                    else:
                        issue_gather(kc + 1, nb)
                compute_chunk(kc, b)

            issue_gather(0, 0)
            process(0, 0, True, False)
            process(1, 1, True, False)
            process(2, 2, True, False)
            process(3, 3, True, True)
            process(4, 0, True, True)

            @pl.loop(5, NCHUNK - 3, step=4)
            def _(kc):
                process(kc, 1, True, True)
                process(kc + 1, 2, True, True)
                process(kc + 2, 3, True, True)
                process(kc + 3, 0, True, True, guard_next=True)

            for s in (2, 3, 0):
                drain_scatter(s)

            plsc.subcore_barrier()
            pltpu.sync_copy(h_sh.at[pl.ds(col0, CPT)],
                            hp_hbm.at[cid, half, pl.ds(col0, CPT)])
            plsc.subcore_barrier()

        pltpu.sync_copy(pden_v, pden_hbm.at[wid])

    return k(z0, z1, srcv, dstv, e_arr, pmax)


def _finalize(hp, pden):
    BR = 2048

    def body(h_ref, d_ref, o_ref):
        den = jnp.sum(d_ref[...], axis=0)[:, None] + 1e-9
        for half in range(2):
            val = (h_ref[0, half] + h_ref[1, half]) / den
            o_ref[:, half * FH:(half + 1) * FH] = jnp.where(
                val > 0, val, jnp.exp(val) - 1.0)

    return pl.pallas_call(
        body,
        grid=(NPAD // BR,),
        in_specs=[
            pl.BlockSpec((NC, 2, BR, FH), lambda i: (0, 0, i, 0)),
            pl.BlockSpec((NW, BR), lambda i: (0, i)),
        ],
        out_specs=pl.BlockSpec((BR, F), lambda i: (i, 0)),
        out_shape=jax.ShapeDtypeStruct((NPAD, F), jnp.float32),
    )(hp, pden)


def kernel(mi_sim, me_sim, edge_index, W_mi, W_me):
    xall = jnp.concatenate([mi_sim, me_sim], axis=0)
    wstack = jnp.stack([W_mi, W_me])
    z = _compute_z(xall, wstack)
    src = edge_index[0].astype(jnp.int32)
    dst = edge_index[1].astype(jnp.int32)
    e_arr, pmax = _edge_logits(z, src, dst)
    hp, pden = _aggregate(z[:, :FH], z[:, FH:], src, dst, e_arr, pmax)
    return _finalize(hp, pden)[:N]


# speedup vs baseline: 10.8235x; 1.0217x over previous
import functools

import jax
import jax.numpy as jnp
from jax import lax
from jax.experimental import pallas as pl
from jax.experimental.pallas import tpu as pltpu
from jax.experimental.pallas import tpu_sc as plsc

N_MI = 5000
N = 10000
D = 512
F = 128
E = 320000
SLOPE = 0.2

L = 16
NC = 2
NS = 16
NW = NC * NS
NPAD = 10240
EPT = E // NW
CH = 80
NCHUNK = EPT // CH
BPC = CH // L
CPT = NPAD // NS
NEG = -1e30


def _rot_perms():
    iota = lax.iota(jnp.int32, L)
    return [(iota + s) & (L - 1) for s in range(1, L)]


def _take(x, idx):
    return jnp.take_along_axis(x, idx, axis=0, mode="promise_in_bounds")


def _matmul_body(x_ref, w_ref, o_ref):
    o_ref[...] = jnp.dot(
        x_ref[...], w_ref[0],
        preferred_element_type=jnp.float32,
    )


def _compute_z(xall, wstack):
    BR = 1000
    return pl.pallas_call(
        _matmul_body,
        grid=(N // BR,),
        in_specs=[
            pl.BlockSpec((BR, D), lambda i: (i, 0)),
            pl.BlockSpec((1, D, F), lambda i: (i * BR // N_MI, 0, 0)),
        ],
        out_specs=pl.BlockSpec((BR, F), lambda i: (i, 0)),
        out_shape=jax.ShapeDtypeStruct((N, F), jnp.float32),
    )(xall, wstack)


def _edge_logits(z, srcv, dstv):
    mesh = plsc.VectorSubcoreMesh(core_axis_name="c", subcore_axis_name="s")

    @functools.partial(
        pl.kernel,
        out_type=(
            jax.ShapeDtypeStruct((E,), jnp.float32),
            jax.ShapeDtypeStruct((NW, NPAD), jnp.float32),
        ),
        mesh=mesh,
        scratch_types=[
            pltpu.VMEM((EPT,), jnp.int32),
            pltpu.VMEM((EPT,), jnp.int32),
            [pltpu.VMEM((CH, F), jnp.float32)] * 2,
            [pltpu.VMEM((CH, F), jnp.float32)] * 2,
            [pltpu.VMEM((CH,), jnp.float32)] * 2,
            pltpu.VMEM((L, L), jnp.float32),
            pltpu.VMEM((NPAD,), jnp.float32),
            [pltpu.SemaphoreType.DMA] * 2,
            [pltpu.SemaphoreType.DMA] * 2,
        ],
        compiler_params=pltpu.CompilerParams(needs_layout_passes=False),
    )
    def k(z_hbm, src_hbm, dst_hbm, e_hbm, pmax_hbm,
          src_all, dst_all, zs, zd, ebuf, scr, pmax_v, semZ, semE):
        cid = lax.axis_index("c")
        sid = lax.axis_index("s")
        wid = sid * NC + cid
        base = wid * EPT
        perms = _rot_perms()
        iota = lax.iota(jnp.int32, L)

        pltpu.sync_copy(src_hbm.at[pl.ds(base, EPT)], src_all)
        pltpu.sync_copy(dst_hbm.at[pl.ds(base, EPT)], dst_all)

        def init_body(i, carry):
            pmax_v[pl.ds(i * L, L)] = jnp.full((L,), NEG, jnp.float32)
            return carry
        lax.fori_loop(0, NPAD // L, init_body, 0)

        def issue_gather(kc, b):
            loc = kc * CH
            pltpu.async_copy(z_hbm.at[src_all.at[pl.ds(loc, CH)]], zs[b], semZ[b])
            pltpu.async_copy(z_hbm.at[dst_all.at[pl.ds(loc, CH)]], zd[b], semZ[b])

        def wait_gather(kc, b):
            loc = kc * CH
            pltpu.make_async_copy(
                z_hbm.at[src_all.at[pl.ds(loc, CH)]], zs[b], semZ[b]).wait()
            pltpu.make_async_copy(
                z_hbm.at[dst_all.at[pl.ds(loc, CH)]], zd[b], semZ[b]).wait()

        def compute_chunk(kc, b):
            loc = kc * CH

            def block_body(j, bc):
                eb = j * L
                for t in range(L):
                    r = eb + t
                    acc = zs[b][r, pl.ds(0, L)] * zd[b][r, pl.ds(0, L)]
                    for c in range(1, F // L):
                        acc = acc + (zs[b][r, pl.ds(c * L, L)]
                                     * zd[b][r, pl.ds(c * L, L)])
                    scr[t] = acc
                dots = plsc.load_gather(scr, [iota, jnp.full((L,), 0, jnp.int32)])
                for l in range(1, L):
                    dots = dots + plsc.load_gather(
                        scr, [iota, jnp.full((L,), l, jnp.int32)])
                ev = jnp.where(dots > 0, dots, SLOPE * dots)
                ebuf[b][pl.ds(eb, L)] = ev
                key = dst_all[pl.ds(loc + eb, L)]
                gmax = ev
                for perm in perms:
                    kr = _take(key, perm)
                    vr = _take(ev, perm)
                    gmax = jnp.maximum(gmax, jnp.where(kr == key, vr, NEG))
                old = plsc.load_gather(pmax_v, [key])
                plsc.store_scatter(pmax_v, [key], jnp.maximum(old, gmax))
                return bc
            lax.fori_loop(0, BPC, block_body, 0)
            pltpu.async_copy(ebuf[b], e_hbm.at[pl.ds(base + kc * CH, CH)], semE[b])

        def process(kc, b, wait_e, next_gather):
            wait_gather(kc, b)
            if next_gather:
                issue_gather(kc + 1, 1 - b)
            if wait_e:
                pltpu.make_async_copy(
                    ebuf[b], e_hbm.at[pl.ds(0, CH)], semE[b]).wait()
            compute_chunk(kc, b)

        issue_gather(0, 0)
        process(0, 0, False, True)
        process(1, 1, False, True)

        @pl.loop(2, NCHUNK - 1, step=2)
        def _(kc):
            process(kc, 0, True, True)
            process(kc + 1, 1, True, True)

        process(NCHUNK - 1, 0, True, False)
        pltpu.make_async_copy(ebuf[0], e_hbm.at[pl.ds(0, CH)], semE[0]).wait()
        pltpu.make_async_copy(ebuf[1], e_hbm.at[pl.ds(0, CH)], semE[1]).wait()
        pltpu.sync_copy(pmax_v, pmax_hbm.at[wid])

    return k(z, srcv, dstv)


FH = F // 2


def _aggregate(z0, z1, srcv, dstv, e_arr, pmax):
    mesh = plsc.VectorSubcoreMesh(core_axis_name="c", subcore_axis_name="s")

    @functools.partial(
        pl.kernel,
        out_type=(
            jax.ShapeDtypeStruct((NC, 2, NPAD, FH), jnp.float32),
            jax.ShapeDtypeStruct((NW, NPAD), jnp.float32),
        ),
        mesh=mesh,
        scratch_types=[
            pltpu.VMEM((4, CPT), jnp.float32),
            pltpu.VMEM((CPT,), jnp.float32),
            pltpu.VMEM((NPAD,), jnp.float32),
            pltpu.VMEM((NPAD,), jnp.float32),
            pltpu.VMEM((EPT,), jnp.float32),
            pltpu.VMEM((EPT,), jnp.int32),
            pltpu.VMEM((EPT,), jnp.int32),
            [pltpu.VMEM((CH,), jnp.int32)] * 4,
            [pltpu.VMEM((CH, FH), jnp.float32)] * 4,
            pltpu.VMEM((CH, FH), jnp.float32),
            pltpu.VMEM_SHARED((NPAD,), jnp.float32),
            pltpu.VMEM_SHARED((NPAD, FH), jnp.float32),
            [pltpu.SemaphoreType.DMA] * 4,
            [pltpu.SemaphoreType.DMA] * 4,
        ],
        compiler_params=pltpu.CompilerParams(
            needs_layout_passes=False, use_tc_tiling_on_sc=False),
    )
    def k(z0_hbm, z1_hbm, src_hbm, dst_hbm, e_hbm, pmax_hbm, hp_hbm, pden_hbm,
          mtmp, msl, mx_full, pden_v, exal, src_all, dst_all, sdidx,
          zs, zbuf, mx_sh, h_sh, semZ, semA):
        cid = lax.axis_index("c")
        sid = lax.axis_index("s")
        wid = sid * NC + cid
        base = wid * EPT
        col0 = sid * CPT
        perms = _rot_perms()

        pltpu.sync_copy(src_hbm.at[pl.ds(base, EPT)], src_all)
        pltpu.sync_copy(dst_hbm.at[pl.ds(base, EPT)], dst_all)
        pltpu.sync_copy(e_hbm.at[pl.ds(base, EPT)], exal)

        for r0 in range(0, NW, 4):
            pltpu.sync_copy(pmax_hbm.at[pl.ds(r0, 4), pl.ds(col0, CPT)], mtmp)

            def mred(v, carry):
                m = mtmp[0, pl.ds(v * L, L)]
                for r in range(1, 4):
                    m = jnp.maximum(m, mtmp[r, pl.ds(v * L, L)])
                if r0 > 0:
                    m = jnp.maximum(m, msl[pl.ds(v * L, L)])
                msl[pl.ds(v * L, L)] = m
                return carry
            lax.fori_loop(0, CPT // L, mred, 0)
        pltpu.sync_copy(msl, mx_sh.at[pl.ds(col0, CPT)])

        def zrow(i, carry):
            for c in range(FH // L):
                zbuf[i, pl.ds(c * L, L)] = jnp.zeros((L,), jnp.float32)
            return carry
        lax.fori_loop(0, CH, zrow, 0)

        def zden(i, carry):
            pden_v[pl.ds(i * L, L)] = jnp.zeros((L,), jnp.float32)
            return carry
        lax.fori_loop(0, NPAD // L, zden, 0)

        plsc.subcore_barrier()
        pltpu.sync_copy(mx_sh, mx_full)

        for half in range(2):
            zh_hbm = z0_hbm if half == 0 else z1_hbm
            zcps = [pltpu.async_copy(
                zbuf, h_sh.at[pl.ds(col0 + b * CH, CH)], semZ[b % 4])
                for b in range(CPT // CH)]
            for cp in zcps:
                cp.wait()
            plsc.subcore_barrier()

            def issue_gather(kc, b):
                loc = kc * CH
                pltpu.async_copy(
                    zh_hbm.at[src_all.at[pl.ds(loc, CH)]], zs[b], semZ[b])

            def wait_gather(kc, b):
                loc = kc * CH
                pltpu.make_async_copy(
                    zh_hbm.at[src_all.at[pl.ds(loc, CH)]], zs[b], semZ[b]).wait()

            def drain_scatter(b):
                pltpu.make_async_copy(
                    zs[b], h_sh.at[sdidx[b]], semA[b]).wait()

            def compute_chunk(kc, b):
                loc = kc * CH
                for v in range(CH // L):
                    sdidx[b][pl.ds(v * L, L)] = dst_all[pl.ds(loc + v * L, L)]

                def block_body(j, bc):
                    eb = j * L
                    key = dst_all[pl.ds(loc + eb, L)]
                    if half == 0:
                        ev = exal[pl.ds(loc + eb, L)]
                        mx = plsc.load_gather(mx_full, [key])
                        ex = jnp.exp(ev - mx)
                        exal[pl.ds(loc + eb, L)] = ex
                        tot = ex
                        for perm in perms:
                            kr = _take(key, perm)
                            vr = _take(ex, perm)
                            tot = tot + jnp.where(kr == key, vr, 0.0)
                        old = plsc.load_gather(pden_v, [key])
                        plsc.store_scatter(pden_v, [key], old + tot)
                    else:
                        ex = exal[pl.ds(loc + eb, L)]
                    for t in range(L):
                        r = eb + t
                        al = _take(ex, jnp.full((L,), t, jnp.int32))
                        for c in range(FH // L):
                            zs[b][r, pl.ds(c * L, L)] = (
                                zs[b][r, pl.ds(c * L, L)] * al)
                    return bc
                lax.fori_loop(0, BPC, block_body, 0)
                pltpu.async_copy(zs[b], h_sh.at[sdidx[b]], semA[b], add=True)

            def process(kc, b, next_gather, drain_a, guard_next=False):
                wait_gather(kc, b)
                nb = (b + 1) % 4
                if next_gather:
                    if drain_a:
                        drain_scatter(nb)
                    if guard_next:
                        @pl.when(kc + 1 < NCHUNK)
                        def _():
                            issue_gather(kc + 1, nb)
                    else:
                        issue_gather(kc + 1, nb)
                compute_chunk(kc, b)

            issue_gather(0, 0)
            process(0, 0, True, False)
            process(1, 1, True, False)
            process(2, 2, True, False)
            process(3, 3, True, True)
            process(4, 0, True, True)

            @pl.loop(5, NCHUNK - 3, step=4)
            def _(kc):
                process(kc, 1, True, True)
                process(kc + 1, 2, True, True)
                process(kc + 2, 3, True, True)
                process(kc + 3, 0, True, True, guard_next=True)

            for s in (2, 3, 0):
                drain_scatter(s)

            plsc.subcore_barrier()
            pltpu.sync_copy(h_sh.at[pl.ds(col0, CPT)],
                            hp_hbm.at[cid, half, pl.ds(col0, CPT)])
            plsc.subcore_barrier()

        pltpu.sync_copy(pden_v, pden_hbm.at[wid])

    return k(z0, z1, srcv, dstv, e_arr, pmax)


def _finalize(hp, pden):
    BR = 2048

    def body(h_ref, d_ref, o_ref):
        den = jnp.sum(d_ref[...], axis=0)[:, None] + 1e-9
        for half in range(2):
            val = (h_ref[0, half] + h_ref[1, half]) / den
            o_ref[:, half * FH:(half + 1) * FH] = jnp.where(
                val > 0, val, jnp.exp(val) - 1.0)

    return pl.pallas_call(
        body,
        grid=(NPAD // BR,),
        in_specs=[
            pl.BlockSpec((NC, 2, BR, FH), lambda i: (0, 0, i, 0)),
            pl.BlockSpec((NW, BR), lambda i: (0, i)),
        ],
        out_specs=pl.BlockSpec((BR, F), lambda i: (i, 0)),
        out_shape=jax.ShapeDtypeStruct((NPAD, F), jnp.float32),
    )(hp, pden)


def kernel(mi_sim, me_sim, edge_index, W_mi, W_me):
    xall = jnp.concatenate([mi_sim, me_sim], axis=0)
    wstack = jnp.stack([W_mi, W_me])
    z = _compute_z(xall, wstack)
    src = edge_index[0].astype(jnp.int32)
    dst = edge_index[1].astype(jnp.int32)
    e_arr, pmax = _edge_logits(z, src, dst)
    hp, pden = _aggregate(z[:, :FH], z[:, FH:], src, dst, e_arr, pmax)
    return _finalize(hp, pden)[:N]
